# baseline (speedup 1.0000x reference)
import functools

import jax
import jax.numpy as jnp
from jax import lax
from jax.experimental import pallas as pl
from jax.experimental.pallas import tpu as pltpu
from jax.experimental.pallas import tpu_sc as plsc

B = 16384
EMB = 16
HID = 32
CH = 128


def _make_gather():
    info = plsc.get_sparse_core_info()
    nw = info.num_cores * info.num_subcores
    b_per_w = B // nw
    n_ch = b_per_w // CH
    mesh = plsc.VectorSubcoreMesh(core_axis_name="c", subcore_axis_name="s")

    @functools.partial(
        pl.kernel,
        mesh=mesh,
        out_type=[
            jax.ShapeDtypeStruct((B, EMB), jnp.float32),
            jax.ShapeDtypeStruct((B, EMB), jnp.float32),
        ],
        scratch_types=[
            pltpu.VMEM((n_ch, CH), jnp.int32),
            pltpu.VMEM((n_ch, CH), jnp.int32),
            pltpu.VMEM((b_per_w, EMB), jnp.float32),
            pltpu.VMEM((b_per_w, EMB), jnp.float32),
            pltpu.SemaphoreType.DMA,
        ],
        compiler_params=pltpu.CompilerParams(use_tc_tiling_on_sc=False),
    )
    def gather(uidx_hbm, qidx_hbm, utab_hbm, qtab_hbm, uout_hbm, qout_hbm,
               uidx_v, qidx_v, urows_v, qrows_v, sem):
        wid = lax.axis_index("s") * info.num_cores + lax.axis_index("c")
        base = wid * b_per_w
        pltpu.sync_copy(uidx_hbm.at[pl.ds(wid * n_ch, n_ch)], uidx_v)
        pltpu.sync_copy(qidx_hbm.at[pl.ds(wid * n_ch, n_ch)], qidx_v)
        copies = []
        for j in range(n_ch):
            copies.append(pltpu.async_copy(
                utab_hbm.at[uidx_v.at[j]], urows_v.at[pl.ds(j * CH, CH)], sem))
            copies.append(pltpu.async_copy(
                qtab_hbm.at[qidx_v.at[j]], qrows_v.at[pl.ds(j * CH, CH)], sem))
        for c in copies:
            c.wait()
        pltpu.sync_copy(urows_v, uout_hbm.at[pl.ds(base, b_per_w)])
        pltpu.sync_copy(qrows_v, qout_hbm.at[pl.ds(base, b_per_w)])

    return gather


_gather = _make_gather()


def _mlp_body(u_ref, q_ref, t_ref, w1u_ref, w1q_ref, w1t_ref, b1_ref,
              w2_ref, b2_ref, o_ref):
    h = (jnp.dot(u_ref[...], w1u_ref[...], preferred_element_type=jnp.float32)
         + jnp.dot(q_ref[...], w1q_ref[...], preferred_element_type=jnp.float32)
         + t_ref[...] * w1t_ref[...]
         + b1_ref[...])
    h = jnp.maximum(h, 0.0)
    o = jnp.dot(h, w2_ref[...], preferred_element_type=jnp.float32) + b2_ref[...]
    o_ref[...] = jax.nn.sigmoid(o)


_mlp = pl.pallas_call(
    _mlp_body,
    out_shape=jax.ShapeDtypeStruct((B, 1), jnp.float32),
)


def kernel(user, quiz, time, user_table, quiz_table, W1, b1, W2, b2):
    uidx = user.astype(jnp.int32).reshape(B // CH, CH)
    qidx = quiz.astype(jnp.int32).reshape(B // CH, CH)
    u, q = _gather(uidx, qidx, user_table, quiz_table)
    out = _mlp(u, q, time,
               W1[:EMB], W1[EMB:2 * EMB], W1[2 * EMB:],
               b1.reshape(1, HID), W2, b2.reshape(1, 1))
    return out.reshape(B)
